# Initial kernel scaffold; baseline (speedup 1.0000x reference)
import jax
import jax.numpy as jnp
from jax.experimental import pallas as pl


def kernel(p0, p1, p2, W, b, C, r):
    raise NotImplementedError("write your pallas kernel here")


# speedup vs baseline: 29.0158x; 29.0158x over previous
import functools

import jax
import jax.numpy as jnp
from jax import lax
from jax.experimental import pallas as pl
from jax.experimental.pallas import tpu as pltpu

_K_ATT = 3
_K_TOT = 6
_NU = 0.001
_ALPHA = 0.1


def _avg_pool3x3(x):
    s = lax.reduce_window(x, 0.0, lax.add, (1, 1, 3, 3), (1, 1, 1, 1),
                          ((0, 0), (0, 0), (1, 1), (1, 1)))
    return s / 9.0


def _descriptor_rows(p0, p1, p2):
    o0 = _avg_pool3x3(p0)
    o1 = _avg_pool3x3(p1)
    o2 = _avg_pool3x3(p2)
    B = o0.shape[0]
    H = o0.shape[2]
    o1 = jax.image.resize(o1, (o1.shape[0], o1.shape[1], H, H), method='bilinear')
    o2 = jax.image.resize(o2, (o2.shape[0], o2.shape[1], H, H), method='bilinear')
    sample = jnp.concatenate([o0, o1, o2], axis=1)
    xx = (jnp.arange(H, dtype=jnp.float32) / (H - 1)) * 2.0 - 1.0
    yy = (jnp.arange(H, dtype=jnp.float32) / (H - 1)) * 2.0 - 1.0
    xx = jnp.broadcast_to(xx[None, None, :, None], (B, 1, H, H))
    yy = jnp.broadcast_to(yy[None, None, None, :], (B, 1, H, H))
    out = jnp.concatenate([sample, xx, yy], axis=1)
    rows = jnp.transpose(out, (0, 2, 3, 1)).reshape(B * H * H, -1)
    return rows


def _topk_body(rows_ref, wt_ref, b_ref, c_ref, out_ref, cn_ref):
    i = pl.program_id(0)

    @pl.when(i == 0)
    def _():
        c = c_ref[...]
        cn_ref[...] = jnp.sum(c * c, axis=0, keepdims=True)

    phi = jnp.dot(rows_ref[...], wt_ref[...],
                  preferred_element_type=jnp.float32,
                  precision=lax.Precision.HIGHEST) + b_ref[...]
    f = jnp.sum(phi * phi, axis=1, keepdims=True)
    fc = jnp.dot(phi, c_ref[...],
                 preferred_element_type=jnp.float32,
                 precision=lax.Precision.HIGHEST)
    d = (f + cn_ref[...]) - 2.0 * fc
    iota = lax.broadcasted_iota(jnp.int32, d.shape, 1)
    for k in range(_K_TOT):
        m = jnp.min(d, axis=1, keepdims=True)
        out_ref[:, k:k + 1] = m
        if k + 1 < _K_TOT:
            idx = jnp.min(jnp.where(d == m, iota, jnp.int32(2 ** 30)),
                          axis=1, keepdims=True)
            d = jnp.where(iota == idx, jnp.float32(jnp.inf), d)


@functools.partial(jax.jit, static_argnames=("rows_per_block",))
def _top6(rows, wt, b2, C, rows_per_block=256):
    n_rows, d_in = rows.shape
    d_out, m = C.shape
    grid = n_rows // rows_per_block
    return pl.pallas_call(
        _topk_body,
        grid=(grid,),
        in_specs=[
            pl.BlockSpec((rows_per_block, d_in), lambda i: (i, 0)),
            pl.BlockSpec((d_in, d_out), lambda i: (0, 0)),
            pl.BlockSpec((1, d_out), lambda i: (0, 0)),
            pl.BlockSpec((d_out, m), lambda i: (0, 0)),
        ],
        out_specs=pl.BlockSpec((rows_per_block, 8), lambda i: (i, 0)),
        out_shape=jax.ShapeDtypeStruct((n_rows, 8), jnp.float32),
        scratch_shapes=[pltpu.VMEM((1, m), jnp.float32)],
    )(rows, wt, b2, C)


def kernel(p0, p1, p2, W, b, C, r):
    B = p0.shape[0]
    scale = p0.shape[2]
    n = scale * scale
    rows = _descriptor_rows(p0, p1, p2)
    top6 = _top6(rows, W.T, b.reshape(1, -1), C)
    d2top = top6[:, :_K_TOT].reshape(B, n, _K_TOT)
    topv = jnp.sqrt(d2top[:, :, :_K_ATT])
    sm = jax.nn.softmax(-topv, axis=-1)
    score = (sm[:, :, 0] * topv[:, :, 0]).reshape(B, scale, scale)[:, None, :, :]
    r2 = r ** 2
    s_att = d2top[:, :, :_K_ATT] - r2
    l_att = (1.0 / _NU) * jnp.mean(jnp.maximum(0.0, s_att))
    s_rep = r2 - d2top[:, :, _K_ATT:]
    l_rep = (1.0 / _NU) * jnp.mean(jnp.maximum(0.0, s_rep - _ALPHA))
    loss = l_att + l_rep
    return (loss, score)


# speedup vs baseline: 35.9474x; 1.2389x over previous
import functools

import jax
import jax.numpy as jnp
from jax import lax
from jax.experimental import pallas as pl
from jax.experimental.pallas import tpu as pltpu

_K_ATT = 3
_K_TOT = 6
_NU = 0.001
_ALPHA = 0.1


def _avg_pool3x3(x):
    s = lax.reduce_window(x, 0.0, lax.add, (1, 1, 3, 3), (1, 1, 1, 1),
                          ((0, 0), (0, 0), (1, 1), (1, 1)))
    return s / 9.0


def _descriptor_rows(p0, p1, p2):
    o0 = _avg_pool3x3(p0)
    o1 = _avg_pool3x3(p1)
    o2 = _avg_pool3x3(p2)
    B = o0.shape[0]
    H = o0.shape[2]
    o1 = jax.image.resize(o1, (o1.shape[0], o1.shape[1], H, H), method='bilinear')
    o2 = jax.image.resize(o2, (o2.shape[0], o2.shape[1], H, H), method='bilinear')
    sample = jnp.concatenate([o0, o1, o2], axis=1)
    xx = (jnp.arange(H, dtype=jnp.float32) / (H - 1)) * 2.0 - 1.0
    yy = (jnp.arange(H, dtype=jnp.float32) / (H - 1)) * 2.0 - 1.0
    xx = jnp.broadcast_to(xx[None, None, :, None], (B, 1, H, H))
    yy = jnp.broadcast_to(yy[None, None, None, :], (B, 1, H, H))
    out = jnp.concatenate([sample, xx, yy], axis=1)
    rows = jnp.transpose(out, (0, 2, 3, 1)).reshape(B * H * H, -1)
    return rows


def _topk_body(rows_ref, wt_ref, b_ref, c_ref, out_ref, cn_ref, chi_ref, clo_ref):
    i = pl.program_id(0)

    @pl.when(i == 0)
    def _():
        c = c_ref[...]
        cn_ref[...] = jnp.sum(c * c, axis=0, keepdims=True)
        chi = c.astype(jnp.bfloat16)
        chi_ref[...] = chi
        clo_ref[...] = (c - chi.astype(jnp.float32)).astype(jnp.bfloat16)

    phi = jnp.dot(rows_ref[...], wt_ref[...],
                  preferred_element_type=jnp.float32,
                  precision=lax.Precision.HIGHEST) + b_ref[...]
    f = jnp.sum(phi * phi, axis=1, keepdims=True)
    phi_hi = phi.astype(jnp.bfloat16)
    phi_lo = (phi - phi_hi.astype(jnp.float32)).astype(jnp.bfloat16)
    chi = chi_ref[...]
    fc = (jnp.dot(phi_hi, chi, preferred_element_type=jnp.float32)
          + jnp.dot(phi_hi, clo_ref[...], preferred_element_type=jnp.float32)
          + jnp.dot(phi_lo, chi, preferred_element_type=jnp.float32))
    d = (f + cn_ref[...]) - 2.0 * fc
    iota = lax.broadcasted_iota(jnp.int32, d.shape, 1)
    for k in range(_K_TOT):
        m = jnp.min(d, axis=1, keepdims=True)
        out_ref[:, k:k + 1] = m
        if k + 1 < _K_TOT:
            idx = jnp.min(jnp.where(d == m, iota, jnp.int32(2 ** 30)),
                          axis=1, keepdims=True)
            d = jnp.where(iota == idx, jnp.float32(jnp.inf), d)


@functools.partial(jax.jit, static_argnames=("rows_per_block",))
def _top6(rows, wt, b2, C, rows_per_block=256):
    n_rows, d_in = rows.shape
    d_out, m = C.shape
    grid = n_rows // rows_per_block
    return pl.pallas_call(
        _topk_body,
        grid=(grid,),
        in_specs=[
            pl.BlockSpec((rows_per_block, d_in), lambda i: (i, 0)),
            pl.BlockSpec((d_in, d_out), lambda i: (0, 0)),
            pl.BlockSpec((1, d_out), lambda i: (0, 0)),
            pl.BlockSpec((d_out, m), lambda i: (0, 0)),
        ],
        out_specs=pl.BlockSpec((rows_per_block, 8), lambda i: (i, 0)),
        out_shape=jax.ShapeDtypeStruct((n_rows, 8), jnp.float32),
        scratch_shapes=[pltpu.VMEM((1, m), jnp.float32),
                        pltpu.VMEM((d_out, m), jnp.bfloat16),
                        pltpu.VMEM((d_out, m), jnp.bfloat16)],
    )(rows, wt, b2, C)


def kernel(p0, p1, p2, W, b, C, r):
    B = p0.shape[0]
    scale = p0.shape[2]
    n = scale * scale
    rows = _descriptor_rows(p0, p1, p2)
    top6 = _top6(rows, W.T, b.reshape(1, -1), C)
    d2top = top6[:, :_K_TOT].reshape(B, n, _K_TOT)
    topv = jnp.sqrt(d2top[:, :, :_K_ATT])
    sm = jax.nn.softmax(-topv, axis=-1)
    score = (sm[:, :, 0] * topv[:, :, 0]).reshape(B, scale, scale)[:, None, :, :]
    r2 = r ** 2
    s_att = d2top[:, :, :_K_ATT] - r2
    l_att = (1.0 / _NU) * jnp.mean(jnp.maximum(0.0, s_att))
    s_rep = r2 - d2top[:, :, _K_ATT:]
    l_rep = (1.0 / _NU) * jnp.mean(jnp.maximum(0.0, s_rep - _ALPHA))
    loss = l_att + l_rep
    return (loss, score)


# speedup vs baseline: 44.4735x; 1.2372x over previous
import functools

import jax
import jax.numpy as jnp
from jax import lax
from jax.experimental import pallas as pl
from jax.experimental.pallas import tpu as pltpu

_K_ATT = 3
_K_TOT = 6
_NU = 0.001
_ALPHA = 0.1


def _avg_pool3x3(x):
    s = lax.reduce_window(x, 0.0, lax.add, (1, 1, 3, 3), (1, 1, 1, 1),
                          ((0, 0), (0, 0), (1, 1), (1, 1)))
    return s / 9.0


def _descriptor_rows(p0, p1, p2):
    o0 = _avg_pool3x3(p0)
    o1 = _avg_pool3x3(p1)
    o2 = _avg_pool3x3(p2)
    B = o0.shape[0]
    H = o0.shape[2]
    o1 = jax.image.resize(o1, (o1.shape[0], o1.shape[1], H, H), method='bilinear')
    o2 = jax.image.resize(o2, (o2.shape[0], o2.shape[1], H, H), method='bilinear')
    sample = jnp.concatenate([o0, o1, o2], axis=1)
    xx = (jnp.arange(H, dtype=jnp.float32) / (H - 1)) * 2.0 - 1.0
    yy = (jnp.arange(H, dtype=jnp.float32) / (H - 1)) * 2.0 - 1.0
    xx = jnp.broadcast_to(xx[None, None, :, None], (B, 1, H, H))
    yy = jnp.broadcast_to(yy[None, None, None, :], (B, 1, H, H))
    out = jnp.concatenate([sample, xx, yy], axis=1)
    rows = jnp.transpose(out, (0, 2, 3, 1)).reshape(B * H * H, -1)
    return rows


def _topk_body(rows_ref, wt_ref, b_ref, c_ref, out_ref, cn_ref, chi_ref, clo_ref):
    i = pl.program_id(0)

    @pl.when(i == 0)
    def _():
        c = c_ref[...]
        cn_ref[...] = jnp.sum(c * c, axis=0, keepdims=True)
        chi = c.astype(jnp.bfloat16)
        chi_ref[...] = chi
        clo_ref[...] = (c - chi.astype(jnp.float32)).astype(jnp.bfloat16)

    phi = jnp.dot(rows_ref[...], wt_ref[...],
                  preferred_element_type=jnp.float32,
                  precision=lax.Precision.HIGHEST) + b_ref[...]
    f = jnp.sum(phi * phi, axis=1, keepdims=True)
    phi_hi = phi.astype(jnp.bfloat16)
    phi_lo = (phi - phi_hi.astype(jnp.float32)).astype(jnp.bfloat16)
    chi = chi_ref[...]
    fc = (jnp.dot(phi_hi, chi, preferred_element_type=jnp.float32)
          + jnp.dot(phi_hi, clo_ref[...], preferred_element_type=jnp.float32)
          + jnp.dot(phi_lo, chi, preferred_element_type=jnp.float32))
    r_blk, m_cols = fc.shape
    cw = 128
    n_chunks = -(-m_cols // cw)
    inf = jnp.float32(jnp.inf)
    cn = cn_ref[...]
    t = [jnp.full((r_blk, cw), inf, jnp.float32) for _ in range(_K_TOT)]
    for c in range(n_chunks):
        lo_c = c * cw
        w = min(cw, m_cols - lo_c)
        x = (f + cn[:, lo_c:lo_c + w]) - 2.0 * fc[:, lo_c:lo_c + w]
        if w < cw:
            x = jnp.concatenate(
                [x, jnp.full((r_blk, cw - w), inf, jnp.float32)], axis=1)
        for j in range(_K_TOT):
            lo = jnp.minimum(t[j], x)
            x = jnp.maximum(t[j], x)
            t[j] = lo
    cand = jnp.concatenate(t, axis=1)
    iota = lax.broadcasted_iota(jnp.int32, cand.shape, 1)
    for k in range(_K_TOT):
        m = jnp.min(cand, axis=1, keepdims=True)
        out_ref[:, k:k + 1] = m
        if k + 1 < _K_TOT:
            idx = jnp.min(jnp.where(cand == m, iota, jnp.int32(2 ** 30)),
                          axis=1, keepdims=True)
            cand = jnp.where(iota == idx, inf, cand)


@functools.partial(jax.jit, static_argnames=("rows_per_block",))
def _top6(rows, wt, b2, C, rows_per_block=256):
    n_rows, d_in = rows.shape
    d_out, m = C.shape
    grid = n_rows // rows_per_block
    return pl.pallas_call(
        _topk_body,
        grid=(grid,),
        in_specs=[
            pl.BlockSpec((rows_per_block, d_in), lambda i: (i, 0)),
            pl.BlockSpec((d_in, d_out), lambda i: (0, 0)),
            pl.BlockSpec((1, d_out), lambda i: (0, 0)),
            pl.BlockSpec((d_out, m), lambda i: (0, 0)),
        ],
        out_specs=pl.BlockSpec((rows_per_block, 8), lambda i: (i, 0)),
        out_shape=jax.ShapeDtypeStruct((n_rows, 8), jnp.float32),
        scratch_shapes=[pltpu.VMEM((1, m), jnp.float32),
                        pltpu.VMEM((d_out, m), jnp.bfloat16),
                        pltpu.VMEM((d_out, m), jnp.bfloat16)],
    )(rows, wt, b2, C)


def kernel(p0, p1, p2, W, b, C, r):
    B = p0.shape[0]
    scale = p0.shape[2]
    n = scale * scale
    rows = _descriptor_rows(p0, p1, p2)
    top6 = _top6(rows, W.T, b.reshape(1, -1), C)
    d2top = top6[:, :_K_TOT].reshape(B, n, _K_TOT)
    topv = jnp.sqrt(d2top[:, :, :_K_ATT])
    sm = jax.nn.softmax(-topv, axis=-1)
    score = (sm[:, :, 0] * topv[:, :, 0]).reshape(B, scale, scale)[:, None, :, :]
    r2 = r ** 2
    s_att = d2top[:, :, :_K_ATT] - r2
    l_att = (1.0 / _NU) * jnp.mean(jnp.maximum(0.0, s_att))
    s_rep = r2 - d2top[:, :, _K_ATT:]
    l_rep = (1.0 / _NU) * jnp.mean(jnp.maximum(0.0, s_rep - _ALPHA))
    loss = l_att + l_rep
    return (loss, score)


# speedup vs baseline: 44.5911x; 1.0026x over previous
import functools

import jax
import jax.numpy as jnp
from jax import lax
from jax.experimental import pallas as pl
from jax.experimental.pallas import tpu as pltpu

_K_ATT = 3
_K_TOT = 6
_NU = 0.001
_ALPHA = 0.1


def _avg_pool3x3(x):
    s = lax.reduce_window(x, 0.0, lax.add, (1, 1, 3, 3), (1, 1, 1, 1),
                          ((0, 0), (0, 0), (1, 1), (1, 1)))
    return s / 9.0


def _descriptor_rows(p0, p1, p2):
    o0 = _avg_pool3x3(p0)
    o1 = _avg_pool3x3(p1)
    o2 = _avg_pool3x3(p2)
    B = o0.shape[0]
    H = o0.shape[2]
    o1 = jax.image.resize(o1, (o1.shape[0], o1.shape[1], H, H), method='bilinear')
    o2 = jax.image.resize(o2, (o2.shape[0], o2.shape[1], H, H), method='bilinear')
    sample = jnp.concatenate([o0, o1, o2], axis=1)
    xx = (jnp.arange(H, dtype=jnp.float32) / (H - 1)) * 2.0 - 1.0
    yy = (jnp.arange(H, dtype=jnp.float32) / (H - 1)) * 2.0 - 1.0
    xx = jnp.broadcast_to(xx[None, None, :, None], (B, 1, H, H))
    yy = jnp.broadcast_to(yy[None, None, None, :], (B, 1, H, H))
    out = jnp.concatenate([sample, xx, yy], axis=1)
    rows = jnp.transpose(out, (0, 2, 3, 1)).reshape(B * H * H, -1)
    return rows


def _topk_body(rows_ref, wt_ref, b_ref, c_ref, out_ref, cn_ref, chi_ref, clo_ref):
    i = pl.program_id(0)

    @pl.when(i == 0)
    def _():
        c = c_ref[...]
        cn_ref[...] = jnp.sum(c * c, axis=0, keepdims=True)
        chi = c.astype(jnp.bfloat16)
        chi_ref[...] = chi
        clo_ref[...] = (c - chi.astype(jnp.float32)).astype(jnp.bfloat16)

    phi = jnp.dot(rows_ref[...], wt_ref[...],
                  preferred_element_type=jnp.float32,
                  precision=lax.Precision.HIGHEST) + b_ref[...]
    f = jnp.sum(phi * phi, axis=1, keepdims=True)
    phi_hi = phi.astype(jnp.bfloat16)
    phi_lo = (phi - phi_hi.astype(jnp.float32)).astype(jnp.bfloat16)
    r_blk = phi.shape[0]
    m_cols = cn_ref.shape[1]
    cw = 128
    scw = 448
    inf = jnp.float32(jnp.inf)
    cn = cn_ref[...]
    t = [jnp.full((r_blk, cw), inf, jnp.float32) for _ in range(_K_TOT)]
    for s0 in range(0, m_cols, scw):
        sw = min(scw, m_cols - s0)
        chi = chi_ref[:, s0:s0 + sw]
        fc = (jnp.dot(phi_hi, chi, preferred_element_type=jnp.float32)
              + jnp.dot(phi_hi, clo_ref[:, s0:s0 + sw],
                        preferred_element_type=jnp.float32)
              + jnp.dot(phi_lo, chi, preferred_element_type=jnp.float32))
        for c in range(0, sw, cw):
            w = min(cw, sw - c)
            x = (f + cn[:, s0 + c:s0 + c + w]) - 2.0 * fc[:, c:c + w]
            if w < cw:
                x = jnp.concatenate(
                    [x, jnp.full((r_blk, cw - w), inf, jnp.float32)], axis=1)
            for j in range(_K_TOT):
                lo = jnp.minimum(t[j], x)
                x = jnp.maximum(t[j], x)
                t[j] = lo
    cand = jnp.concatenate(t, axis=1)
    iota = lax.broadcasted_iota(jnp.int32, cand.shape, 1)
    for k in range(_K_TOT):
        m = jnp.min(cand, axis=1, keepdims=True)
        out_ref[:, k:k + 1] = m
        if k + 1 < _K_TOT:
            idx = jnp.min(jnp.where(cand == m, iota, jnp.int32(2 ** 30)),
                          axis=1, keepdims=True)
            cand = jnp.where(iota == idx, inf, cand)


@functools.partial(jax.jit, static_argnames=("rows_per_block",))
def _top6(rows, wt, b2, C, rows_per_block=256):
    n_rows, d_in = rows.shape
    d_out, m = C.shape
    grid = n_rows // rows_per_block
    return pl.pallas_call(
        _topk_body,
        grid=(grid,),
        in_specs=[
            pl.BlockSpec((rows_per_block, d_in), lambda i: (i, 0)),
            pl.BlockSpec((d_in, d_out), lambda i: (0, 0)),
            pl.BlockSpec((1, d_out), lambda i: (0, 0)),
            pl.BlockSpec((d_out, m), lambda i: (0, 0)),
        ],
        out_specs=pl.BlockSpec((rows_per_block, 8), lambda i: (i, 0)),
        out_shape=jax.ShapeDtypeStruct((n_rows, 8), jnp.float32),
        scratch_shapes=[pltpu.VMEM((1, m), jnp.float32),
                        pltpu.VMEM((d_out, m), jnp.bfloat16),
                        pltpu.VMEM((d_out, m), jnp.bfloat16)],
    )(rows, wt, b2, C)


def kernel(p0, p1, p2, W, b, C, r):
    B = p0.shape[0]
    scale = p0.shape[2]
    n = scale * scale
    rows = _descriptor_rows(p0, p1, p2)
    top6 = _top6(rows, W.T, b.reshape(1, -1), C)
    d2top = top6[:, :_K_TOT].reshape(B, n, _K_TOT)
    topv = jnp.sqrt(d2top[:, :, :_K_ATT])
    sm = jax.nn.softmax(-topv, axis=-1)
    score = (sm[:, :, 0] * topv[:, :, 0]).reshape(B, scale, scale)[:, None, :, :]
    r2 = r ** 2
    s_att = d2top[:, :, :_K_ATT] - r2
    l_att = (1.0 / _NU) * jnp.mean(jnp.maximum(0.0, s_att))
    s_rep = r2 - d2top[:, :, _K_ATT:]
    l_rep = (1.0 / _NU) * jnp.mean(jnp.maximum(0.0, s_rep - _ALPHA))
    loss = l_att + l_rep
    return (loss, score)


# speedup vs baseline: 49.4072x; 1.1080x over previous
import functools

import jax
import jax.numpy as jnp
from jax import lax
from jax.experimental import pallas as pl
from jax.experimental.pallas import tpu as pltpu

_K_ATT = 3
_K_TOT = 6
_NU = 0.001
_ALPHA = 0.1


def _avg_pool3x3(x):
    s = lax.reduce_window(x, 0.0, lax.add, (1, 1, 3, 3), (1, 1, 1, 1),
                          ((0, 0), (0, 0), (1, 1), (1, 1)))
    return s / 9.0


def _pool_mat(n):
    ii = jnp.arange(n)[:, None]
    jj = jnp.arange(n)[None, :]
    return (jnp.abs(ii - jj) <= 1).astype(jnp.float32) / 3.0


def _spatial_op(n_out, n_in):
    a = _pool_mat(n_in)
    if n_out != n_in:
        r = jax.image.resize(jnp.eye(n_in, dtype=jnp.float32),
                             (n_out, n_in), method='bilinear')
        a = r @ a
    return a


def _descriptor_rows(p0, p1, p2):
    B = p0.shape[0]
    H = p0.shape[2]
    mats = [_spatial_op(H, p.shape[2]) for p in (p0, p1, p2)]
    outs = [jnp.einsum('hH,bcHW,wW->bchw', a, p, a,
                       precision=lax.Precision.HIGHEST)
            for a, p in zip(mats, (p0, p1, p2))]
    sample = jnp.concatenate(outs, axis=1)
    xx = (jnp.arange(H, dtype=jnp.float32) / (H - 1)) * 2.0 - 1.0
    yy = (jnp.arange(H, dtype=jnp.float32) / (H - 1)) * 2.0 - 1.0
    xx = jnp.broadcast_to(xx[None, None, :, None], (B, 1, H, H))
    yy = jnp.broadcast_to(yy[None, None, None, :], (B, 1, H, H))
    out = jnp.concatenate([sample, xx, yy], axis=1)
    rows = jnp.transpose(out, (0, 2, 3, 1)).reshape(B * H * H, -1)
    return rows


def _topk_body(rows_ref, wt_ref, b_ref, c_ref, out_ref, cn_ref, chi_ref, clo_ref):
    i = pl.program_id(0)

    @pl.when(i == 0)
    def _():
        c = c_ref[...]
        cn_ref[...] = jnp.sum(c * c, axis=0, keepdims=True)
        chi = c.astype(jnp.bfloat16)
        chi_ref[...] = chi
        clo_ref[...] = (c - chi.astype(jnp.float32)).astype(jnp.bfloat16)

    phi = jnp.dot(rows_ref[...], wt_ref[...],
                  preferred_element_type=jnp.float32,
                  precision=lax.Precision.HIGHEST) + b_ref[...]
    f = jnp.sum(phi * phi, axis=1, keepdims=True)
    phi_hi = phi.astype(jnp.bfloat16)
    phi_lo = (phi - phi_hi.astype(jnp.float32)).astype(jnp.bfloat16)
    r_blk = phi.shape[0]
    m_cols = cn_ref.shape[1]
    cw = 128
    scw = 448
    inf = jnp.float32(jnp.inf)
    cn = cn_ref[...]
    t = [jnp.full((r_blk, cw), inf, jnp.float32) for _ in range(_K_TOT)]
    for s0 in range(0, m_cols, scw):
        sw = min(scw, m_cols - s0)
        chi = chi_ref[:, s0:s0 + sw]
        fc = (jnp.dot(phi_hi, chi, preferred_element_type=jnp.float32)
              + jnp.dot(phi_hi, clo_ref[:, s0:s0 + sw],
                        preferred_element_type=jnp.float32)
              + jnp.dot(phi_lo, chi, preferred_element_type=jnp.float32))
        for c in range(0, sw, cw):
            w = min(cw, sw - c)
            x = (f + cn[:, s0 + c:s0 + c + w]) - 2.0 * fc[:, c:c + w]
            if w < cw:
                x = jnp.concatenate(
                    [x, jnp.full((r_blk, cw - w), inf, jnp.float32)], axis=1)
            for j in range(_K_TOT):
                lo = jnp.minimum(t[j], x)
                x = jnp.maximum(t[j], x)
                t[j] = lo
    cand = jnp.concatenate(t, axis=1)
    iota = lax.broadcasted_iota(jnp.int32, cand.shape, 1)
    for k in range(_K_TOT):
        m = jnp.min(cand, axis=1, keepdims=True)
        out_ref[:, k:k + 1] = m
        if k + 1 < _K_TOT:
            idx = jnp.min(jnp.where(cand == m, iota, jnp.int32(2 ** 30)),
                          axis=1, keepdims=True)
            cand = jnp.where(iota == idx, inf, cand)


@functools.partial(jax.jit, static_argnames=("rows_per_block",))
def _top6(rows, wt, b2, C, rows_per_block=256):
    n_rows, d_in = rows.shape
    d_out, m = C.shape
    grid = n_rows // rows_per_block
    return pl.pallas_call(
        _topk_body,
        grid=(grid,),
        in_specs=[
            pl.BlockSpec((rows_per_block, d_in), lambda i: (i, 0)),
            pl.BlockSpec((d_in, d_out), lambda i: (0, 0)),
            pl.BlockSpec((1, d_out), lambda i: (0, 0)),
            pl.BlockSpec((d_out, m), lambda i: (0, 0)),
        ],
        out_specs=pl.BlockSpec((rows_per_block, 8), lambda i: (i, 0)),
        out_shape=jax.ShapeDtypeStruct((n_rows, 8), jnp.float32),
        scratch_shapes=[pltpu.VMEM((1, m), jnp.float32),
                        pltpu.VMEM((d_out, m), jnp.bfloat16),
                        pltpu.VMEM((d_out, m), jnp.bfloat16)],
    )(rows, wt, b2, C)


def kernel(p0, p1, p2, W, b, C, r):
    B = p0.shape[0]
    scale = p0.shape[2]
    n = scale * scale
    rows = _descriptor_rows(p0, p1, p2)
    top6 = _top6(rows, W.T, b.reshape(1, -1), C)
    d2top = top6[:, :_K_TOT].reshape(B, n, _K_TOT)
    topv = jnp.sqrt(d2top[:, :, :_K_ATT])
    sm = jax.nn.softmax(-topv, axis=-1)
    score = (sm[:, :, 0] * topv[:, :, 0]).reshape(B, scale, scale)[:, None, :, :]
    r2 = r ** 2
    s_att = d2top[:, :, :_K_ATT] - r2
    l_att = (1.0 / _NU) * jnp.mean(jnp.maximum(0.0, s_att))
    s_rep = r2 - d2top[:, :, _K_ATT:]
    l_rep = (1.0 / _NU) * jnp.mean(jnp.maximum(0.0, s_rep - _ALPHA))
    loss = l_att + l_rep
    return (loss, score)


# speedup vs baseline: 49.4888x; 1.0017x over previous
import functools

import jax
import jax.numpy as jnp
from jax import lax
from jax.experimental import pallas as pl
from jax.experimental.pallas import tpu as pltpu

_K_ATT = 3
_K_TOT = 6
_NU = 0.001
_ALPHA = 0.1


def _avg_pool3x3(x):
    s = lax.reduce_window(x, 0.0, lax.add, (1, 1, 3, 3), (1, 1, 1, 1),
                          ((0, 0), (0, 0), (1, 1), (1, 1)))
    return s / 9.0


def _pool_mat(n):
    ii = jnp.arange(n)[:, None]
    jj = jnp.arange(n)[None, :]
    return (jnp.abs(ii - jj) <= 1).astype(jnp.float32) / 3.0


def _spatial_op(n_out, n_in):
    a = _pool_mat(n_in)
    if n_out != n_in:
        r = jax.image.resize(jnp.eye(n_in, dtype=jnp.float32),
                             (n_out, n_in), method='bilinear')
        a = r @ a
    return a


def _descriptor_rows(p0, p1, p2):
    B = p0.shape[0]
    H = p0.shape[2]
    mats = [_spatial_op(H, p.shape[2]) for p in (p0, p1, p2)]
    outs = [jnp.einsum('hH,bcHW,wW->bchw', a, p, a,
                       precision=lax.Precision.HIGHEST)
            for a, p in zip(mats, (p0, p1, p2))]
    sample = jnp.concatenate(outs, axis=1)
    xx = (jnp.arange(H, dtype=jnp.float32) / (H - 1)) * 2.0 - 1.0
    yy = (jnp.arange(H, dtype=jnp.float32) / (H - 1)) * 2.0 - 1.0
    xx = jnp.broadcast_to(xx[None, None, :, None], (B, 1, H, H))
    yy = jnp.broadcast_to(yy[None, None, None, :], (B, 1, H, H))
    out = jnp.concatenate([sample, xx, yy], axis=1)
    rows = jnp.transpose(out, (0, 2, 3, 1)).reshape(B * H * H, -1)
    return rows


def _topk_body(rows_ref, wt_ref, b_ref, c_ref, out_ref, cn_ref, chi_ref, clo_ref):
    i = pl.program_id(0)

    @pl.when(i == 0)
    def _():
        c = c_ref[...]
        cn_ref[...] = jnp.sum(c * c, axis=0, keepdims=True)
        chi = c.astype(jnp.bfloat16)
        chi_ref[...] = chi
        clo_ref[...] = (c - chi.astype(jnp.float32)).astype(jnp.bfloat16)

    phi = jnp.dot(rows_ref[...], wt_ref[...],
                  preferred_element_type=jnp.float32,
                  precision=lax.Precision.HIGHEST) + b_ref[...]
    f = jnp.sum(phi * phi, axis=1, keepdims=True)
    phi_hi = phi.astype(jnp.bfloat16)
    phi_lo = (phi - phi_hi.astype(jnp.float32)).astype(jnp.bfloat16)
    chi = chi_ref[...]
    fc = (jnp.dot(phi_hi, chi, preferred_element_type=jnp.float32)
          + jnp.dot(phi_hi, clo_ref[...], preferred_element_type=jnp.float32)
          + jnp.dot(phi_lo, chi, preferred_element_type=jnp.float32))
    r_blk = phi.shape[0]
    m_cols = cn_ref.shape[1]
    cw = 128
    rsb = 64
    inf = jnp.float32(jnp.inf)
    cn = cn_ref[...]
    for r0 in range(0, r_blk, rsb):
        f_sb = f[r0:r0 + rsb]
        t = [jnp.full((rsb, cw), inf, jnp.float32) for _ in range(_K_TOT)]
        for c in range(0, m_cols, cw):
            w = min(cw, m_cols - c)
            x = (f_sb + cn[:, c:c + w]) - 2.0 * fc[r0:r0 + rsb, c:c + w]
            if w < cw:
                x = jnp.concatenate(
                    [x, jnp.full((rsb, cw - w), inf, jnp.float32)], axis=1)
            for j in range(_K_TOT):
                lo = jnp.minimum(t[j], x)
                x = jnp.maximum(t[j], x)
                t[j] = lo
        cand = jnp.concatenate(t, axis=1)
        iota = lax.broadcasted_iota(jnp.int32, cand.shape, 1)
        for k in range(_K_TOT):
            m = jnp.min(cand, axis=1, keepdims=True)
            out_ref[r0:r0 + rsb, k:k + 1] = m
            if k + 1 < _K_TOT:
                idx = jnp.min(jnp.where(cand == m, iota, jnp.int32(2 ** 30)),
                              axis=1, keepdims=True)
                cand = jnp.where(iota == idx, inf, cand)


@functools.partial(jax.jit, static_argnames=("rows_per_block",))
def _top6(rows, wt, b2, C, rows_per_block=256):
    n_rows, d_in = rows.shape
    d_out, m = C.shape
    grid = n_rows // rows_per_block
    return pl.pallas_call(
        _topk_body,
        grid=(grid,),
        in_specs=[
            pl.BlockSpec((rows_per_block, d_in), lambda i: (i, 0)),
            pl.BlockSpec((d_in, d_out), lambda i: (0, 0)),
            pl.BlockSpec((1, d_out), lambda i: (0, 0)),
            pl.BlockSpec((d_out, m), lambda i: (0, 0)),
        ],
        out_specs=pl.BlockSpec((rows_per_block, 8), lambda i: (i, 0)),
        out_shape=jax.ShapeDtypeStruct((n_rows, 8), jnp.float32),
        scratch_shapes=[pltpu.VMEM((1, m), jnp.float32),
                        pltpu.VMEM((d_out, m), jnp.bfloat16),
                        pltpu.VMEM((d_out, m), jnp.bfloat16)],
    )(rows, wt, b2, C)


def kernel(p0, p1, p2, W, b, C, r):
    B = p0.shape[0]
    scale = p0.shape[2]
    n = scale * scale
    rows = _descriptor_rows(p0, p1, p2)
    top6 = _top6(rows, W.T, b.reshape(1, -1), C)
    d2top = top6[:, :_K_TOT].reshape(B, n, _K_TOT)
    topv = jnp.sqrt(d2top[:, :, :_K_ATT])
    sm = jax.nn.softmax(-topv, axis=-1)
    score = (sm[:, :, 0] * topv[:, :, 0]).reshape(B, scale, scale)[:, None, :, :]
    r2 = r ** 2
    s_att = d2top[:, :, :_K_ATT] - r2
    l_att = (1.0 / _NU) * jnp.mean(jnp.maximum(0.0, s_att))
    s_rep = r2 - d2top[:, :, _K_ATT:]
    l_rep = (1.0 / _NU) * jnp.mean(jnp.maximum(0.0, s_rep - _ALPHA))
    loss = l_att + l_rep
    return (loss, score)


# speedup vs baseline: 52.0001x; 1.0507x over previous
import functools

import jax
import jax.numpy as jnp
from jax import lax
from jax.experimental import pallas as pl
from jax.experimental.pallas import tpu as pltpu

_K_ATT = 3
_K_TOT = 6
_NU = 0.001
_ALPHA = 0.1


def _avg_pool3x3(x):
    s = lax.reduce_window(x, 0.0, lax.add, (1, 1, 3, 3), (1, 1, 1, 1),
                          ((0, 0), (0, 0), (1, 1), (1, 1)))
    return s / 9.0


def _pool_mat(n):
    ii = jnp.arange(n)[:, None]
    jj = jnp.arange(n)[None, :]
    return (jnp.abs(ii - jj) <= 1).astype(jnp.float32) / 3.0


def _spatial_op(n_out, n_in):
    a = _pool_mat(n_in)
    if n_out != n_in:
        r = jax.image.resize(jnp.eye(n_in, dtype=jnp.float32),
                             (n_out, n_in), method='bilinear')
        a = r @ a
    return a


def _descriptor_rows(p0, p1, p2):
    B = p0.shape[0]
    H = p0.shape[2]
    mats = [_spatial_op(H, p.shape[2]) for p in (p0, p1, p2)]
    outs = [jnp.einsum('hH,bcHW,wW->bchw', a, p, a,
                       precision=lax.Precision.HIGHEST)
            for a, p in zip(mats, (p0, p1, p2))]
    sample = jnp.concatenate(outs, axis=1)
    xx = (jnp.arange(H, dtype=jnp.float32) / (H - 1)) * 2.0 - 1.0
    yy = (jnp.arange(H, dtype=jnp.float32) / (H - 1)) * 2.0 - 1.0
    xx = jnp.broadcast_to(xx[None, None, :, None], (B, 1, H, H))
    yy = jnp.broadcast_to(yy[None, None, None, :], (B, 1, H, H))
    out = jnp.concatenate([sample, xx, yy], axis=1)
    rows = jnp.transpose(out, (0, 2, 3, 1)).reshape(B * H * H, -1)
    return rows


def _topk_body(rows_ref, wt_ref, b_ref, c_ref, out_ref, cn_ref, chi_ref,
               clo_ref, wthi_ref, wtlo_ref):
    i = pl.program_id(0)

    @pl.when(i == 0)
    def _():
        c = c_ref[...]
        cn_ref[...] = jnp.sum(c * c, axis=0, keepdims=True)
        chi = c.astype(jnp.bfloat16)
        chi_ref[...] = chi
        clo_ref[...] = (c - chi.astype(jnp.float32)).astype(jnp.bfloat16)
        wt = wt_ref[...]
        wthi = wt.astype(jnp.bfloat16)
        wthi_ref[...] = wthi
        wtlo_ref[...] = (wt - wthi.astype(jnp.float32)).astype(jnp.bfloat16)

    rows = rows_ref[...]
    rows_hi = rows.astype(jnp.bfloat16)
    rows_lo = (rows - rows_hi.astype(jnp.float32)).astype(jnp.bfloat16)
    wt_hi = wthi_ref[...]
    phi = (jnp.dot(rows_hi, wt_hi, preferred_element_type=jnp.float32)
           + jnp.dot(rows_hi, wtlo_ref[...], preferred_element_type=jnp.float32)
           + jnp.dot(rows_lo, wt_hi, preferred_element_type=jnp.float32)
           ) + b_ref[...]
    f = jnp.sum(phi * phi, axis=1, keepdims=True)
    phi_hi = phi.astype(jnp.bfloat16)
    phi_lo = (phi - phi_hi.astype(jnp.float32)).astype(jnp.bfloat16)
    chi = chi_ref[...]
    fc = (jnp.dot(phi_hi, chi, preferred_element_type=jnp.float32)
          + jnp.dot(phi_hi, clo_ref[...], preferred_element_type=jnp.float32)
          + jnp.dot(phi_lo, chi, preferred_element_type=jnp.float32))
    r_blk = phi.shape[0]
    m_cols = cn_ref.shape[1]
    cw = 128
    rsb = 64
    inf = jnp.float32(jnp.inf)
    cn = cn_ref[...]
    for r0 in range(0, r_blk, rsb):
        f_sb = f[r0:r0 + rsb]
        t = [jnp.full((rsb, cw), inf, jnp.float32) for _ in range(_K_TOT)]
        for c in range(0, m_cols, cw):
            w = min(cw, m_cols - c)
            x = (f_sb + cn[:, c:c + w]) - 2.0 * fc[r0:r0 + rsb, c:c + w]
            if w < cw:
                x = jnp.concatenate(
                    [x, jnp.full((rsb, cw - w), inf, jnp.float32)], axis=1)
            for j in range(_K_TOT):
                lo = jnp.minimum(t[j], x)
                x = jnp.maximum(t[j], x)
                t[j] = lo
        cand = jnp.concatenate(t, axis=1)
        iota = lax.broadcasted_iota(jnp.int32, cand.shape, 1)
        for k in range(_K_TOT):
            m = jnp.min(cand, axis=1, keepdims=True)
            out_ref[r0:r0 + rsb, k:k + 1] = m
            if k + 1 < _K_TOT:
                idx = jnp.min(jnp.where(cand == m, iota, jnp.int32(2 ** 30)),
                              axis=1, keepdims=True)
                cand = jnp.where(iota == idx, inf, cand)


@functools.partial(jax.jit, static_argnames=("rows_per_block",))
def _top6(rows, wt, b2, C, rows_per_block=256):
    n_rows, d_in = rows.shape
    d_out, m = C.shape
    grid = n_rows // rows_per_block
    return pl.pallas_call(
        _topk_body,
        grid=(grid,),
        in_specs=[
            pl.BlockSpec((rows_per_block, d_in), lambda i: (i, 0)),
            pl.BlockSpec((d_in, d_out), lambda i: (0, 0)),
            pl.BlockSpec((1, d_out), lambda i: (0, 0)),
            pl.BlockSpec((d_out, m), lambda i: (0, 0)),
        ],
        out_specs=pl.BlockSpec((rows_per_block, 8), lambda i: (i, 0)),
        out_shape=jax.ShapeDtypeStruct((n_rows, 8), jnp.float32),
        scratch_shapes=[pltpu.VMEM((1, m), jnp.float32),
                        pltpu.VMEM((d_out, m), jnp.bfloat16),
                        pltpu.VMEM((d_out, m), jnp.bfloat16),
                        pltpu.VMEM((d_in, d_out), jnp.bfloat16),
                        pltpu.VMEM((d_in, d_out), jnp.bfloat16)],
    )(rows, wt, b2, C)


def kernel(p0, p1, p2, W, b, C, r):
    B = p0.shape[0]
    scale = p0.shape[2]
    n = scale * scale
    rows = _descriptor_rows(p0, p1, p2)
    top6 = _top6(rows, W.T, b.reshape(1, -1), C)
    d2top = top6[:, :_K_TOT].reshape(B, n, _K_TOT)
    topv = jnp.sqrt(d2top[:, :, :_K_ATT])
    sm = jax.nn.softmax(-topv, axis=-1)
    score = (sm[:, :, 0] * topv[:, :, 0]).reshape(B, scale, scale)[:, None, :, :]
    r2 = r ** 2
    s_att = d2top[:, :, :_K_ATT] - r2
    l_att = (1.0 / _NU) * jnp.mean(jnp.maximum(0.0, s_att))
    s_rep = r2 - d2top[:, :, _K_ATT:]
    l_rep = (1.0 / _NU) * jnp.mean(jnp.maximum(0.0, s_rep - _ALPHA))
    loss = l_att + l_rep
    return (loss, score)


# speedup vs baseline: 52.1982x; 1.0038x over previous
import functools

import jax
import jax.numpy as jnp
from jax import lax
from jax.experimental import pallas as pl
from jax.experimental.pallas import tpu as pltpu

_K_ATT = 3
_K_TOT = 6
_NU = 0.001
_ALPHA = 0.1


def _avg_pool3x3(x):
    s = lax.reduce_window(x, 0.0, lax.add, (1, 1, 3, 3), (1, 1, 1, 1),
                          ((0, 0), (0, 0), (1, 1), (1, 1)))
    return s / 9.0


def _pool_mat(n):
    ii = jnp.arange(n)[:, None]
    jj = jnp.arange(n)[None, :]
    return (jnp.abs(ii - jj) <= 1).astype(jnp.float32) / 3.0


def _spatial_op(n_out, n_in):
    a = _pool_mat(n_in)
    if n_out != n_in:
        r = jax.image.resize(jnp.eye(n_in, dtype=jnp.float32),
                             (n_out, n_in), method='bilinear')
        a = r @ a
    return a


def _descriptor_rows(p0, p1, p2):
    B = p0.shape[0]
    H = p0.shape[2]
    mats = [_spatial_op(H, p.shape[2]) for p in (p0, p1, p2)]
    outs = [jnp.einsum('hH,bcHW,wW->bhwc', a, p, a,
                       precision=lax.Precision.HIGHEST)
            for a, p in zip(mats, (p0, p1, p2))]
    sample = jnp.concatenate(outs, axis=3)
    cv = (jnp.arange(H, dtype=jnp.float32) / (H - 1)) * 2.0 - 1.0
    xx = jnp.broadcast_to(cv[None, :, None, None], (B, H, H, 1))
    yy = jnp.broadcast_to(cv[None, None, :, None], (B, H, H, 1))
    out = jnp.concatenate([sample, xx, yy], axis=3)
    rows = out.reshape(B * H * H, -1)
    return rows


def _topk_body(rows_ref, wt_ref, b_ref, c_ref, out_ref, cn_ref, chi_ref,
               clo_ref, wthi_ref, wtlo_ref):
    i = pl.program_id(0)

    @pl.when(i == 0)
    def _():
        c = c_ref[...]
        cn_ref[...] = jnp.sum(c * c, axis=0, keepdims=True)
        chi = c.astype(jnp.bfloat16)
        chi_ref[...] = chi
        clo_ref[...] = (c - chi.astype(jnp.float32)).astype(jnp.bfloat16)
        wt = wt_ref[...]
        wthi = wt.astype(jnp.bfloat16)
        wthi_ref[...] = wthi
        wtlo_ref[...] = (wt - wthi.astype(jnp.float32)).astype(jnp.bfloat16)

    rows = rows_ref[...]
    rows_hi = rows.astype(jnp.bfloat16)
    rows_lo = (rows - rows_hi.astype(jnp.float32)).astype(jnp.bfloat16)
    wt_hi = wthi_ref[...]
    phi = (jnp.dot(rows_hi, wt_hi, preferred_element_type=jnp.float32)
           + jnp.dot(rows_hi, wtlo_ref[...], preferred_element_type=jnp.float32)
           + jnp.dot(rows_lo, wt_hi, preferred_element_type=jnp.float32)
           ) + b_ref[...]
    f = jnp.sum(phi * phi, axis=1, keepdims=True)
    phi_hi = phi.astype(jnp.bfloat16)
    phi_lo = (phi - phi_hi.astype(jnp.float32)).astype(jnp.bfloat16)
    chi = chi_ref[...]
    fc = (jnp.dot(phi_hi, chi, preferred_element_type=jnp.float32)
          + jnp.dot(phi_hi, clo_ref[...], preferred_element_type=jnp.float32)
          + jnp.dot(phi_lo, chi, preferred_element_type=jnp.float32))
    r_blk = phi.shape[0]
    m_cols = cn_ref.shape[1]
    cw = 128
    rsb = 64
    inf = jnp.float32(jnp.inf)
    cn = cn_ref[...]
    for r0 in range(0, r_blk, rsb):
        f_sb = f[r0:r0 + rsb]
        t = [jnp.full((rsb, cw), inf, jnp.float32) for _ in range(_K_TOT)]
        for c in range(0, m_cols, cw):
            w = min(cw, m_cols - c)
            x = (f_sb + cn[:, c:c + w]) - 2.0 * fc[r0:r0 + rsb, c:c + w]
            if w < cw:
                x = jnp.concatenate(
                    [x, jnp.full((rsb, cw - w), inf, jnp.float32)], axis=1)
            for j in range(_K_TOT):
                lo = jnp.minimum(t[j], x)
                x = jnp.maximum(t[j], x)
                t[j] = lo
        cand = jnp.concatenate(t, axis=1)
        iota = lax.broadcasted_iota(jnp.int32, cand.shape, 1)
        for k in range(_K_TOT):
            m = jnp.min(cand, axis=1, keepdims=True)
            out_ref[r0:r0 + rsb, k:k + 1] = m
            if k + 1 < _K_TOT:
                idx = jnp.min(jnp.where(cand == m, iota, jnp.int32(2 ** 30)),
                              axis=1, keepdims=True)
                cand = jnp.where(iota == idx, inf, cand)


@functools.partial(jax.jit, static_argnames=("rows_per_block",))
def _top6(rows, wt, b2, C, rows_per_block=256):
    n_rows, d_in = rows.shape
    d_out, m = C.shape
    grid = n_rows // rows_per_block
    return pl.pallas_call(
        _topk_body,
        grid=(grid,),
        in_specs=[
            pl.BlockSpec((rows_per_block, d_in), lambda i: (i, 0)),
            pl.BlockSpec((d_in, d_out), lambda i: (0, 0)),
            pl.BlockSpec((1, d_out), lambda i: (0, 0)),
            pl.BlockSpec((d_out, m), lambda i: (0, 0)),
        ],
        out_specs=pl.BlockSpec((rows_per_block, 8), lambda i: (i, 0)),
        out_shape=jax.ShapeDtypeStruct((n_rows, 8), jnp.float32),
        scratch_shapes=[pltpu.VMEM((1, m), jnp.float32),
                        pltpu.VMEM((d_out, m), jnp.bfloat16),
                        pltpu.VMEM((d_out, m), jnp.bfloat16),
                        pltpu.VMEM((d_in, d_out), jnp.bfloat16),
                        pltpu.VMEM((d_in, d_out), jnp.bfloat16)],
    )(rows, wt, b2, C)


def kernel(p0, p1, p2, W, b, C, r):
    B = p0.shape[0]
    scale = p0.shape[2]
    n = scale * scale
    rows = _descriptor_rows(p0, p1, p2)
    top6 = _top6(rows, W.T, b.reshape(1, -1), C)
    d2top = top6[:, :_K_TOT].reshape(B, n, _K_TOT)
    topv = jnp.sqrt(d2top[:, :, :_K_ATT])
    sm = jax.nn.softmax(-topv, axis=-1)
    score = (sm[:, :, 0] * topv[:, :, 0]).reshape(B, scale, scale)[:, None, :, :]
    r2 = r ** 2
    s_att = d2top[:, :, :_K_ATT] - r2
    l_att = (1.0 / _NU) * jnp.mean(jnp.maximum(0.0, s_att))
    s_rep = r2 - d2top[:, :, _K_ATT:]
    l_rep = (1.0 / _NU) * jnp.mean(jnp.maximum(0.0, s_rep - _ALPHA))
    loss = l_att + l_rep
    return (loss, score)


# speedup vs baseline: 56.3081x; 1.0787x over previous
import functools

import jax
import jax.numpy as jnp
from jax import lax
from jax.experimental import pallas as pl
from jax.experimental.pallas import tpu as pltpu

_K_ATT = 3
_K_TOT = 6
_NU = 0.001
_ALPHA = 0.1


def _avg_pool3x3(x):
    s = lax.reduce_window(x, 0.0, lax.add, (1, 1, 3, 3), (1, 1, 1, 1),
                          ((0, 0), (0, 0), (1, 1), (1, 1)))
    return s / 9.0


def _pool_mat(n):
    ii = jnp.arange(n)[:, None]
    jj = jnp.arange(n)[None, :]
    return (jnp.abs(ii - jj) <= 1).astype(jnp.float32) / 3.0


def _spatial_op(n_out, n_in):
    a = _pool_mat(n_in)
    if n_out != n_in:
        r = jax.image.resize(jnp.eye(n_in, dtype=jnp.float32),
                             (n_out, n_in), method='bilinear')
        a = r @ a
    return a


def _descriptor_rows(p0, p1, p2):
    B = p0.shape[0]
    H = p0.shape[2]
    mats = [_spatial_op(H, p.shape[2]) for p in (p0, p1, p2)]
    outs = [jnp.einsum('hH,bcHW,wW->bhwc', a, p, a,
                       precision=lax.Precision.HIGH)
            for a, p in zip(mats, (p0, p1, p2))]
    sample = jnp.concatenate(outs, axis=3)
    cv = (jnp.arange(H, dtype=jnp.float32) / (H - 1)) * 2.0 - 1.0
    xx = jnp.broadcast_to(cv[None, :, None, None], (B, H, H, 1))
    yy = jnp.broadcast_to(cv[None, None, :, None], (B, H, H, 1))
    out = jnp.concatenate([sample, xx, yy], axis=3)
    rows = out.reshape(B * H * H, -1)
    return rows


def _topk_body(rows_ref, wt_ref, b_ref, c_ref, out_ref, cn_ref, chi_ref,
               clo_ref, wthi_ref, wtlo_ref):
    i = pl.program_id(0)

    @pl.when(i == 0)
    def _():
        c = c_ref[...]
        cn_ref[...] = jnp.sum(c * c, axis=0, keepdims=True)
        chi = c.astype(jnp.bfloat16)
        chi_ref[...] = chi
        clo_ref[...] = (c - chi.astype(jnp.float32)).astype(jnp.bfloat16)
        wt = wt_ref[...]
        wthi = wt.astype(jnp.bfloat16)
        wthi_ref[...] = wthi
        wtlo_ref[...] = (wt - wthi.astype(jnp.float32)).astype(jnp.bfloat16)

    rows = rows_ref[...]
    rows_hi = rows.astype(jnp.bfloat16)
    rows_lo = (rows - rows_hi.astype(jnp.float32)).astype(jnp.bfloat16)
    wt_hi = wthi_ref[...]
    phi = (jnp.dot(rows_hi, wt_hi, preferred_element_type=jnp.float32)
           + jnp.dot(rows_hi, wtlo_ref[...], preferred_element_type=jnp.float32)
           + jnp.dot(rows_lo, wt_hi, preferred_element_type=jnp.float32)
           ) + b_ref[...]
    f = jnp.sum(phi * phi, axis=1, keepdims=True)
    phi_hi = phi.astype(jnp.bfloat16)
    phi_lo = (phi - phi_hi.astype(jnp.float32)).astype(jnp.bfloat16)
    chi = chi_ref[...]
    fc = (jnp.dot(phi_hi, chi, preferred_element_type=jnp.float32)
          + jnp.dot(phi_hi, clo_ref[...], preferred_element_type=jnp.float32)
          + jnp.dot(phi_lo, chi, preferred_element_type=jnp.float32))
    r_blk = phi.shape[0]
    m_cols = cn_ref.shape[1]
    cw = 128
    rsb = 64
    inf = jnp.float32(jnp.inf)
    cn = cn_ref[...]
    for r0 in range(0, r_blk, rsb):
        f_sb = f[r0:r0 + rsb]
        t = [jnp.full((rsb, cw), inf, jnp.float32) for _ in range(_K_TOT)]
        for c in range(0, m_cols, cw):
            w = min(cw, m_cols - c)
            x = (f_sb + cn[:, c:c + w]) - 2.0 * fc[r0:r0 + rsb, c:c + w]
            if w < cw:
                x = jnp.concatenate(
                    [x, jnp.full((rsb, cw - w), inf, jnp.float32)], axis=1)
            for j in range(_K_TOT):
                lo = jnp.minimum(t[j], x)
                x = jnp.maximum(t[j], x)
                t[j] = lo
        cand = jnp.concatenate(t, axis=1)
        iota = lax.broadcasted_iota(jnp.int32, cand.shape, 1)
        for k in range(_K_TOT):
            m = jnp.min(cand, axis=1, keepdims=True)
            out_ref[r0:r0 + rsb, k:k + 1] = m
            if k + 1 < _K_TOT:
                idx = jnp.min(jnp.where(cand == m, iota, jnp.int32(2 ** 30)),
                              axis=1, keepdims=True)
                cand = jnp.where(iota == idx, inf, cand)


@functools.partial(jax.jit, static_argnames=("rows_per_block",))
def _top6(rows, wt, b2, C, rows_per_block=448):
    n_rows, d_in = rows.shape
    d_out, m = C.shape
    grid = n_rows // rows_per_block
    return pl.pallas_call(
        _topk_body,
        grid=(grid,),
        in_specs=[
            pl.BlockSpec((rows_per_block, d_in), lambda i: (i, 0)),
            pl.BlockSpec((d_in, d_out), lambda i: (0, 0)),
            pl.BlockSpec((1, d_out), lambda i: (0, 0)),
            pl.BlockSpec((d_out, m), lambda i: (0, 0)),
        ],
        out_specs=pl.BlockSpec((rows_per_block, 8), lambda i: (i, 0)),
        out_shape=jax.ShapeDtypeStruct((n_rows, 8), jnp.float32),
        scratch_shapes=[pltpu.VMEM((1, m), jnp.float32),
                        pltpu.VMEM((d_out, m), jnp.bfloat16),
                        pltpu.VMEM((d_out, m), jnp.bfloat16),
                        pltpu.VMEM((d_in, d_out), jnp.bfloat16),
                        pltpu.VMEM((d_in, d_out), jnp.bfloat16)],
    )(rows, wt, b2, C)


def kernel(p0, p1, p2, W, b, C, r):
    B = p0.shape[0]
    scale = p0.shape[2]
    n = scale * scale
    rows = _descriptor_rows(p0, p1, p2)
    top6 = _top6(rows, W.T, b.reshape(1, -1), C)
    d2top = top6[:, :_K_TOT].reshape(B, n, _K_TOT)
    topv = jnp.sqrt(d2top[:, :, :_K_ATT])
    sm = jax.nn.softmax(-topv, axis=-1)
    score = (sm[:, :, 0] * topv[:, :, 0]).reshape(B, scale, scale)[:, None, :, :]
    r2 = r ** 2
    s_att = d2top[:, :, :_K_ATT] - r2
    l_att = (1.0 / _NU) * jnp.mean(jnp.maximum(0.0, s_att))
    s_rep = r2 - d2top[:, :, _K_ATT:]
    l_rep = (1.0 / _NU) * jnp.mean(jnp.maximum(0.0, s_rep - _ALPHA))
    loss = l_att + l_rep
    return (loss, score)
